# baseline (speedup 1.0000x reference)
import functools

import jax
import jax.numpy as jnp
from jax import lax
from jax.experimental import pallas as pl
from jax.experimental.pallas import tpu as pltpu
from jax.experimental.pallas import tpu_sc as plsc

N_NODES = 10000
N_EDGES = 320000
D = 128

NC = 2
NS = 16
NW = NC * NS

CH = 80
EPT = N_EDGES // NW
NCHUNK = EPT // CH
NPAD = 10240
ZB = 40
NBLK = N_NODES // ZB
NSEG = 5
SEGC = NCHUNK // NSEG

_mesh = plsc.VectorSubcoreMesh(core_axis_name="c", subcore_axis_name="s")


HR = NPAD // D
RB8 = 8
NRB = HR // RB8


@functools.partial(
    pl.kernel,
    out_type=jax.ShapeDtypeStruct((NC, HR, D), jnp.float32),
    mesh=_mesh,
    scratch_types=[
        pltpu.VMEM((EPT,), jnp.int32),
        pltpu.VMEM((HR, D), jnp.float32),
        pltpu.VMEM((RB8, D), jnp.float32),
        pltpu.VMEM((RB8, D), jnp.float32),
        pltpu.VMEM_SHARED((NS, HR, D), jnp.float32),
    ],
    compiler_params=pltpu.CompilerParams(needs_layout_passes=False),
)
def _sc_deg(dst_hbm, degp_hbm, dstb, hist, racc, rtmp, slots):
    c = lax.axis_index("c")
    s = lax.axis_index("s")
    wid = c * NS + s

    def zf(i, _):
        for k in range(D // 16):
            hist[i, pl.ds(k * 16, 16)] = jnp.zeros((16,), jnp.float32)
        return 0
    lax.fori_loop(0, HR, zf, 0)

    pltpu.sync_copy(dst_hbm.at[pl.ds(wid * EPT, EPT)], dstb)

    ones = jnp.ones((16,), jnp.float32)

    def body(i, _):
        for u in range(5):
            idx = dstb[pl.ds(i * 80 + u * 16, 16)]
            hi = lax.shift_right_logical(idx, 7)
            lo = lax.bitwise_and(idx, jnp.int32(D - 1))
            plsc.addupdate_scatter(hist, [hi, lo], ones)
        return 0
    lax.fori_loop(0, EPT // 80, body, 0)

    pltpu.sync_copy(hist, slots.at[s])
    plsc.subcore_barrier()

    @pl.when(s < NRB)
    def _():
        pltpu.sync_copy(slots.at[0, pl.ds(s * RB8, RB8)], racc)
        for r in range(1, NS):
            pltpu.sync_copy(slots.at[r, pl.ds(s * RB8, RB8)], rtmp)

            def add(i, _):
                for k in range(D // 16):
                    racc[i, pl.ds(k * 16, 16)] = (
                        racc[i, pl.ds(k * 16, 16)] + rtmp[i, pl.ds(k * 16, 16)])
                return 0
            lax.fori_loop(0, RB8, add, 0)
        pltpu.sync_copy(racc, degp_hbm.at[c, pl.ds(s * RB8, RB8)])


@functools.partial(
    pl.kernel,
    out_type=jax.ShapeDtypeStruct((NC, N_NODES, D), jnp.float32),
    mesh=_mesh,
    scratch_types=[
        pltpu.VMEM((SEGC * CH,), jnp.int32),
        pltpu.VMEM((SEGC, CH), jnp.int32),
        pltpu.VMEM((CH, D), jnp.float32),
        pltpu.VMEM((CH, D), jnp.float32),
        pltpu.VMEM((CH, D), jnp.float32),
        pltpu.VMEM((ZB, D), jnp.float32),
        pltpu.VMEM_SHARED((N_NODES, D), jnp.float32),
        pltpu.SemaphoreType.DMA,
        pltpu.SemaphoreType.DMA,
        pltpu.SemaphoreType.DMA,
        pltpu.SemaphoreType.DMA,
        pltpu.SemaphoreType.DMA,
        pltpu.SemaphoreType.DMA,
    ],
    compiler_params=pltpu.CompilerParams(needs_layout_passes=False),
)
def _sc_scatter(hp_hbm, src_hbm, dst_hbm, s_hbm,
                srcb, dstb, rowsA, rowsB, rowsC, stage, acc,
                gA, gB, gC, sA, sB, sC):
    c = lax.axis_index("c")
    s = lax.axis_index("s")
    wid = c * NS + s

    def zfill(i, _):
        for k in range(D // 16):
            stage[i, pl.ds(k * 16, 16)] = jnp.zeros((16,), jnp.float32)
        return 0
    lax.fori_loop(0, ZB, zfill, 0)

    for k in range((NBLK + NS - 1) // NS):
        blk = k * NS + s

        @pl.when(blk < NBLK)
        def _():
            pltpu.sync_copy(stage, acc.at[pl.ds(blk * ZB, ZB)])
    plsc.subcore_barrier()

    def gather(j, buf, sem):
        return pltpu.async_copy(hp_hbm.at[srcb.at[pl.ds(j * CH, CH)]], buf, sem)

    def gwait(buf, sem):
        pltpu.make_async_copy(hp_hbm.at[srcb.at[pl.ds(0, CH)]], buf, sem).wait()

    def scat(j, buf, sem):
        return pltpu.async_copy(buf, acc.at[dstb.at[j]], sem, add=True)

    for seg in range(NSEG):
        pltpu.sync_copy(
            src_hbm.at[pl.ds(wid * EPT + seg * SEGC * CH, SEGC * CH)], srcb)
        pltpu.sync_copy(dst_hbm.at[wid, seg], dstb)

        gather(0, rowsA, gA)
        gather(1, rowsB, gB)
        gather(2, rowsC, gC)

        def body(i, _):
            a = 3 * i
            gwait(rowsA, gA)
            dA = scat(a, rowsA, sA)
            gwait(rowsB, gB)
            dB = scat(a + 1, rowsB, sB)
            gwait(rowsC, gC)
            dC = scat(a + 2, rowsC, sC)
            dA.wait()
            gather(a + 3, rowsA, gA)
            dB.wait()

            @pl.when(i < (SEGC - 1) // 3 - 1)
            def _():
                gather(a + 4, rowsB, gB)
            dC.wait()

            @pl.when(i < (SEGC - 1) // 3 - 1)
            def _():
                gather(a + 5, rowsC, gC)
            return 0
        lax.fori_loop(0, (SEGC - 1) // 3, body, 0)

        gwait(rowsA, gA)
        scat(SEGC - 1, rowsA, sA).wait()

    plsc.subcore_barrier()
    for k in range((NBLK + NS - 1) // NS):
        blk = k * NS + s

        @pl.when(blk < NBLK)
        def _():
            pltpu.sync_copy(acc.at[pl.ds(blk * ZB, ZB)], stage)
            pltpu.sync_copy(stage, s_hbm.at[c, pl.ds(blk * ZB, ZB)])


def _tc_mm_body(x_ref, w_ref, out_ref):
    out_ref[...] = jnp.dot(x_ref[...], w_ref[...],
                           preferred_element_type=jnp.float32)


def _tc_mm(x, W):
    rb = 1000
    return pl.pallas_call(
        _tc_mm_body,
        out_shape=jax.ShapeDtypeStruct((N_NODES, D), jnp.float32),
        grid=(N_NODES // rb,),
        in_specs=[
            pl.BlockSpec((rb, D), lambda i: (i, 0)),
            pl.BlockSpec((D, D), lambda i: (0, 0)),
        ],
        out_specs=pl.BlockSpec((rb, D), lambda i: (i, 0)),
    )(x, W)


def _tc_scale_body(h_ref, degp_ref, out_ref):
    deg = degp_ref[0, :, 0] + degp_ref[1, :, 0] + 1.0
    dinv = lax.rsqrt(deg)
    out_ref[...] = h_ref[...] * dinv[:, None]


def _tc_scale(h, degp):
    rb = 1000
    return pl.pallas_call(
        _tc_scale_body,
        out_shape=jax.ShapeDtypeStruct((N_NODES, D), jnp.float32),
        grid=(N_NODES // rb,),
        in_specs=[
            pl.BlockSpec((rb, D), lambda i: (i, 0)),
            pl.BlockSpec((NC, rb, 1), lambda i: (0, i, 0)),
        ],
        out_specs=pl.BlockSpec((rb, D), lambda i: (i, 0)),
    )(h, degp)


def _tc_final_body(s_ref, hp_ref, x_ref, b_ref, degp_ref, out_ref):
    deg = degp_ref[0, :, 0] + degp_ref[1, :, 0] + 1.0
    dinv = lax.rsqrt(deg)
    tot = s_ref[0] + s_ref[1] + hp_ref[...]
    out_ref[...] = jnp.maximum(tot * dinv[:, None] + b_ref[...] + x_ref[...], 0.0)


def _tc_final(S, hp, x, b2, degp):
    rb = 1000
    return pl.pallas_call(
        _tc_final_body,
        out_shape=jax.ShapeDtypeStruct((N_NODES, D), jnp.float32),
        grid=(N_NODES // rb,),
        in_specs=[
            pl.BlockSpec((NC, rb, D), lambda i: (0, i, 0)),
            pl.BlockSpec((rb, D), lambda i: (i, 0)),
            pl.BlockSpec((rb, D), lambda i: (i, 0)),
            pl.BlockSpec((1, D), lambda i: (0, 0)),
            pl.BlockSpec((NC, rb, 1), lambda i: (0, i, 0)),
        ],
        out_specs=pl.BlockSpec((rb, D), lambda i: (i, 0)),
    )(S, hp, x, b2, degp)


def kernel(x, edge_index, W, b):
    src = edge_index[0].astype(jnp.int32)
    dst = edge_index[1].astype(jnp.int32)

    h = _tc_mm(x, W)
    degp = _sc_deg(dst).reshape(NC, NPAD, 1)
    hp = _tc_scale(h, degp)
    S = _sc_scatter(hp, src, dst.reshape(NW, NSEG, SEGC, CH))
    return _tc_final(S, hp, x, b.reshape(1, D), degp)


# speedup vs baseline: 1.0181x; 1.0181x over previous
import functools

import jax
import jax.numpy as jnp
from jax import lax
from jax.experimental import pallas as pl
from jax.experimental.pallas import tpu as pltpu
from jax.experimental.pallas import tpu_sc as plsc

N_NODES = 10000
N_EDGES = 320000
D = 128

NC = 2
NS = 16
NW = NC * NS

CH = 80
EPT = N_EDGES // NW
NCHUNK = EPT // CH
NPAD = 10240
ZB = 80
NBLK = N_NODES // ZB
NSEG = 5
SEGC = NCHUNK // NSEG

_mesh = plsc.VectorSubcoreMesh(core_axis_name="c", subcore_axis_name="s")


HR = NPAD // D
RB8 = 8
NRB = HR // RB8


@functools.partial(
    pl.kernel,
    out_type=jax.ShapeDtypeStruct((NC, HR, D), jnp.float32),
    mesh=_mesh,
    scratch_types=[
        pltpu.VMEM((EPT,), jnp.int32),
        pltpu.VMEM((HR, D), jnp.float32),
        pltpu.VMEM((RB8, D), jnp.float32),
        pltpu.VMEM((RB8, D), jnp.float32),
        pltpu.VMEM_SHARED((NS, HR, D), jnp.float32),
    ],
    compiler_params=pltpu.CompilerParams(needs_layout_passes=False),
)
def _sc_deg(dst_hbm, degp_hbm, dstb, hist, racc, rtmp, slots):
    c = lax.axis_index("c")
    s = lax.axis_index("s")
    wid = c * NS + s

    def zf(i, _):
        for k in range(D // 16):
            hist[i, pl.ds(k * 16, 16)] = jnp.zeros((16,), jnp.float32)
        return 0
    lax.fori_loop(0, HR, zf, 0)

    pltpu.sync_copy(dst_hbm.at[pl.ds(wid * EPT, EPT)], dstb)

    ones = jnp.ones((16,), jnp.float32)

    def body(i, _):
        for u in range(5):
            idx = dstb[pl.ds(i * 80 + u * 16, 16)]
            hi = lax.shift_right_logical(idx, 7)
            lo = lax.bitwise_and(idx, jnp.int32(D - 1))
            plsc.addupdate_scatter(hist, [hi, lo], ones)
        return 0
    lax.fori_loop(0, EPT // 80, body, 0)

    pltpu.sync_copy(hist, slots.at[s])
    plsc.subcore_barrier()

    @pl.when(s < NRB)
    def _():
        pltpu.sync_copy(slots.at[0, pl.ds(s * RB8, RB8)], racc)
        for r in range(1, NS):
            pltpu.sync_copy(slots.at[r, pl.ds(s * RB8, RB8)], rtmp)

            def add(i, _):
                for k in range(D // 16):
                    racc[i, pl.ds(k * 16, 16)] = (
                        racc[i, pl.ds(k * 16, 16)] + rtmp[i, pl.ds(k * 16, 16)])
                return 0
            lax.fori_loop(0, RB8, add, 0)
        pltpu.sync_copy(racc, degp_hbm.at[c, pl.ds(s * RB8, RB8)])


@functools.partial(
    pl.kernel,
    out_type=jax.ShapeDtypeStruct((NC, N_NODES, D), jnp.float32),
    mesh=_mesh,
    scratch_types=[
        pltpu.VMEM((SEGC * CH,), jnp.int32),
        pltpu.VMEM((SEGC, CH), jnp.int32),
        pltpu.VMEM((CH, D), jnp.float32),
        pltpu.VMEM((CH, D), jnp.float32),
        pltpu.VMEM((CH, D), jnp.float32),
        pltpu.VMEM_SHARED((N_NODES, D), jnp.float32),
        pltpu.SemaphoreType.DMA,
        pltpu.SemaphoreType.DMA,
        pltpu.SemaphoreType.DMA,
        pltpu.SemaphoreType.DMA,
        pltpu.SemaphoreType.DMA,
        pltpu.SemaphoreType.DMA,
    ],
    compiler_params=pltpu.CompilerParams(needs_layout_passes=False),
)
def _sc_scatter(hp_hbm, src_hbm, dst_hbm, s_hbm,
                srcb, dstb, rowsA, rowsB, rowsC, acc,
                gA, gB, gC, sA, sB, sC):
    c = lax.axis_index("c")
    s = lax.axis_index("s")
    wid = c * NS + s
    ktail = NBLK - (NBLK // NS) * NS

    def zfill(i, _):
        for k in range(D // 16):
            rowsC[i, pl.ds(k * 16, 16)] = jnp.zeros((16,), jnp.float32)
        return 0
    lax.fori_loop(0, ZB, zfill, 0)

    for k in range(NBLK // NS):
        pltpu.sync_copy(rowsC, acc.at[pl.ds((k * NS + s) * ZB, ZB)])

    @pl.when(s < ktail)
    def _():
        pltpu.sync_copy(rowsC, acc.at[pl.ds(((NBLK // NS) * NS + s) * ZB, ZB)])
    plsc.subcore_barrier()

    def gather(j, buf, sem):
        return pltpu.async_copy(hp_hbm.at[srcb.at[pl.ds(j * CH, CH)]], buf, sem)

    def gwait(buf, sem):
        pltpu.make_async_copy(hp_hbm.at[srcb.at[pl.ds(0, CH)]], buf, sem).wait()

    def scat(j, buf, sem):
        return pltpu.async_copy(buf, acc.at[dstb.at[j]], sem, add=True)

    for seg in range(NSEG):
        pltpu.sync_copy(
            src_hbm.at[pl.ds(wid * EPT + seg * SEGC * CH, SEGC * CH)], srcb)
        pltpu.sync_copy(dst_hbm.at[wid, seg], dstb)

        gather(0, rowsA, gA)
        gather(1, rowsB, gB)
        gather(2, rowsC, gC)

        def body(i, _):
            a = 3 * i
            gwait(rowsA, gA)
            dA = scat(a, rowsA, sA)
            gwait(rowsB, gB)
            dB = scat(a + 1, rowsB, sB)
            gwait(rowsC, gC)
            dC = scat(a + 2, rowsC, sC)
            dA.wait()
            gather(a + 3, rowsA, gA)
            dB.wait()

            @pl.when(i < (SEGC - 1) // 3 - 1)
            def _():
                gather(a + 4, rowsB, gB)
            dC.wait()

            @pl.when(i < (SEGC - 1) // 3 - 1)
            def _():
                gather(a + 5, rowsC, gC)
            return 0
        lax.fori_loop(0, (SEGC - 1) // 3, body, 0)

        gwait(rowsA, gA)
        scat(SEGC - 1, rowsA, sA).wait()

    plsc.subcore_barrier()
    prev = [None, None]
    for k in range(NBLK // NS):
        b = k % 2
        buf, gsem, ssem = (rowsA, gA, sA) if b == 0 else (rowsB, gB, sB)
        if prev[b] is not None:
            prev[b].wait()
        r0 = (k * NS + s) * ZB
        pltpu.async_copy(acc.at[pl.ds(r0, ZB)], buf, gsem).wait()
        prev[b] = pltpu.async_copy(buf, s_hbm.at[c, pl.ds(r0, ZB)], ssem)
    prev[0].wait()
    prev[1].wait()

    @pl.when(s < ktail)
    def _():
        r0 = ((NBLK // NS) * NS + s) * ZB
        pltpu.async_copy(acc.at[pl.ds(r0, ZB)], rowsA, gA).wait()
        pltpu.async_copy(rowsA, s_hbm.at[c, pl.ds(r0, ZB)], sA).wait()


def _tc_mm_body(x_ref, w_ref, out_ref):
    out_ref[...] = jnp.dot(x_ref[...], w_ref[...],
                           preferred_element_type=jnp.float32)


def _tc_mm(x, W):
    rb = 1000
    return pl.pallas_call(
        _tc_mm_body,
        out_shape=jax.ShapeDtypeStruct((N_NODES, D), jnp.float32),
        grid=(N_NODES // rb,),
        in_specs=[
            pl.BlockSpec((rb, D), lambda i: (i, 0)),
            pl.BlockSpec((D, D), lambda i: (0, 0)),
        ],
        out_specs=pl.BlockSpec((rb, D), lambda i: (i, 0)),
    )(x, W)


def _tc_scale_body(h_ref, degp_ref, out_ref):
    deg = degp_ref[0, :, 0] + degp_ref[1, :, 0] + 1.0
    dinv = lax.rsqrt(deg)
    out_ref[...] = h_ref[...] * dinv[:, None]


def _tc_scale(h, degp):
    rb = 1000
    return pl.pallas_call(
        _tc_scale_body,
        out_shape=jax.ShapeDtypeStruct((N_NODES, D), jnp.float32),
        grid=(N_NODES // rb,),
        in_specs=[
            pl.BlockSpec((rb, D), lambda i: (i, 0)),
            pl.BlockSpec((NC, rb, 1), lambda i: (0, i, 0)),
        ],
        out_specs=pl.BlockSpec((rb, D), lambda i: (i, 0)),
    )(h, degp)


def _tc_final_body(s_ref, hp_ref, x_ref, b_ref, degp_ref, out_ref):
    deg = degp_ref[0, :, 0] + degp_ref[1, :, 0] + 1.0
    dinv = lax.rsqrt(deg)
    tot = s_ref[0] + s_ref[1] + hp_ref[...]
    out_ref[...] = jnp.maximum(tot * dinv[:, None] + b_ref[...] + x_ref[...], 0.0)


def _tc_final(S, hp, x, b2, degp):
    rb = 1000
    return pl.pallas_call(
        _tc_final_body,
        out_shape=jax.ShapeDtypeStruct((N_NODES, D), jnp.float32),
        grid=(N_NODES // rb,),
        in_specs=[
            pl.BlockSpec((NC, rb, D), lambda i: (0, i, 0)),
            pl.BlockSpec((rb, D), lambda i: (i, 0)),
            pl.BlockSpec((rb, D), lambda i: (i, 0)),
            pl.BlockSpec((1, D), lambda i: (0, 0)),
            pl.BlockSpec((NC, rb, 1), lambda i: (0, i, 0)),
        ],
        out_specs=pl.BlockSpec((rb, D), lambda i: (i, 0)),
    )(S, hp, x, b2, degp)


def kernel(x, edge_index, W, b):
    src = edge_index[0].astype(jnp.int32)
    dst = edge_index[1].astype(jnp.int32)

    h = _tc_mm(x, W)
    degp = _sc_deg(dst).reshape(NC, NPAD, 1)
    hp = _tc_scale(h, degp)
    S = _sc_scatter(hp, src, dst.reshape(NW, NSEG, SEGC, CH))
    return _tc_final(S, hp, x, b.reshape(1, D), degp)


# speedup vs baseline: 1.0502x; 1.0315x over previous
import functools

import jax
import jax.numpy as jnp
from jax import lax
from jax.experimental import pallas as pl
from jax.experimental.pallas import tpu as pltpu
from jax.experimental.pallas import tpu_sc as plsc

N_NODES = 10000
N_EDGES = 320000
D = 128

NC = 2
NS = 16
NW = NC * NS

CH = 80
EPT = N_EDGES // NW
NCHUNK = EPT // CH
NPAD = 10240
ZB = 80
NBLK = N_NODES // ZB
NSEG = 5
SEGC = NCHUNK // NSEG

_mesh = plsc.VectorSubcoreMesh(core_axis_name="c", subcore_axis_name="s")


HR = NPAD // D
RB8 = 8
NRB = HR // RB8


@functools.partial(
    pl.kernel,
    out_type=jax.ShapeDtypeStruct((NC, HR, D), jnp.float32),
    mesh=_mesh,
    scratch_types=[
        pltpu.VMEM((EPT,), jnp.int32),
        pltpu.VMEM((HR, D), jnp.float32),
        pltpu.VMEM((RB8, D), jnp.float32),
        pltpu.VMEM((RB8, D), jnp.float32),
        pltpu.VMEM_SHARED((NS, HR, D), jnp.float32),
    ],
    compiler_params=pltpu.CompilerParams(needs_layout_passes=False),
)
def _sc_deg(dst_hbm, degp_hbm, dstb, hist, racc, rtmp, slots):
    c = lax.axis_index("c")
    s = lax.axis_index("s")
    wid = c * NS + s

    def zf(i, _):
        for k in range(D // 16):
            hist[i, pl.ds(k * 16, 16)] = jnp.zeros((16,), jnp.float32)
        return 0
    lax.fori_loop(0, HR, zf, 0)

    pltpu.sync_copy(dst_hbm.at[pl.ds(wid * EPT, EPT)], dstb)

    ones = jnp.ones((16,), jnp.float32)

    def body(i, _):
        for u in range(5):
            idx = dstb[pl.ds(i * 80 + u * 16, 16)]
            hi = lax.shift_right_logical(idx, 7)
            lo = lax.bitwise_and(idx, jnp.int32(D - 1))
            plsc.addupdate_scatter(hist, [hi, lo], ones)
        return 0
    lax.fori_loop(0, EPT // 80, body, 0)

    pltpu.sync_copy(hist, slots.at[s])
    plsc.subcore_barrier()

    @pl.when(s < NRB)
    def _():
        pltpu.sync_copy(slots.at[0, pl.ds(s * RB8, RB8)], racc)
        for r in range(1, NS):
            pltpu.sync_copy(slots.at[r, pl.ds(s * RB8, RB8)], rtmp)

            def add(i, _):
                for k in range(D // 16):
                    racc[i, pl.ds(k * 16, 16)] = (
                        racc[i, pl.ds(k * 16, 16)] + rtmp[i, pl.ds(k * 16, 16)])
                return 0
            lax.fori_loop(0, RB8, add, 0)
        pltpu.sync_copy(racc, degp_hbm.at[c, pl.ds(s * RB8, RB8)])


@functools.partial(
    pl.kernel,
    out_type=jax.ShapeDtypeStruct((NC, N_NODES, D), jnp.float32),
    mesh=_mesh,
    scratch_types=[
        pltpu.VMEM((SEGC * CH,), jnp.int32),
        pltpu.VMEM((SEGC * CH,), jnp.int32),
        pltpu.VMEM((SEGC, CH), jnp.int32),
        pltpu.VMEM((SEGC, CH), jnp.int32),
        pltpu.VMEM((CH, D), jnp.float32),
        pltpu.VMEM((CH, D), jnp.float32),
        pltpu.VMEM((CH, D), jnp.float32),
        pltpu.VMEM_SHARED((N_NODES, D), jnp.float32),
        pltpu.SemaphoreType.DMA,
        pltpu.SemaphoreType.DMA,
        pltpu.SemaphoreType.DMA,
        pltpu.SemaphoreType.DMA,
        pltpu.SemaphoreType.DMA,
        pltpu.SemaphoreType.DMA,
        pltpu.SemaphoreType.DMA,
        pltpu.SemaphoreType.DMA,
    ],
    compiler_params=pltpu.CompilerParams(needs_layout_passes=False),
)
def _sc_scatter(hp_hbm, src_hbm, dst_hbm, s_hbm,
                srcb0, srcb1, dstb0, dstb1, rowsA, rowsB, rowsC, acc,
                gA, gB, gC, sA, sB, sC, iA, iB):
    c = lax.axis_index("c")
    s = lax.axis_index("s")
    wid = c * NS + s
    ktail = NBLK - (NBLK // NS) * NS

    def loadseg(seg, sb, db, sem):
        d1 = pltpu.async_copy(
            src_hbm.at[pl.ds(wid * EPT + seg * SEGC * CH, SEGC * CH)], sb, sem)
        d2 = pltpu.async_copy(dst_hbm.at[wid, seg], db, sem)
        return (d1, d2)

    pend = loadseg(0, srcb0, dstb0, iA)

    def zfill(i, _):
        for k in range(D // 16):
            rowsC[i, pl.ds(k * 16, 16)] = jnp.zeros((16,), jnp.float32)
        return 0
    lax.fori_loop(0, ZB, zfill, 0)

    for k in range(NBLK // NS):
        pltpu.sync_copy(rowsC, acc.at[pl.ds((k * NS + s) * ZB, ZB)])

    @pl.when(s < ktail)
    def _():
        pltpu.sync_copy(rowsC, acc.at[pl.ds(((NBLK // NS) * NS + s) * ZB, ZB)])
    plsc.subcore_barrier()

    def gather(j, sb, buf, sem):
        return pltpu.async_copy(hp_hbm.at[sb.at[pl.ds(j * CH, CH)]], buf, sem)

    def gwait(buf, sem):
        pltpu.make_async_copy(hp_hbm.at[srcb0.at[pl.ds(0, CH)]], buf, sem).wait()

    def scat(j, db, buf, sem):
        return pltpu.async_copy(buf, acc.at[db.at[j]], sem, add=True)

    for seg in range(NSEG):
        sb, db = (srcb0, dstb0) if seg % 2 == 0 else (srcb1, dstb1)
        pend[0].wait()
        pend[1].wait()

        gather(0, sb, rowsA, gA)
        gather(1, sb, rowsB, gB)
        gather(2, sb, rowsC, gC)
        if seg + 1 < NSEG:
            pend = loadseg(seg + 1,
                           srcb1 if seg % 2 == 0 else srcb0,
                           dstb1 if seg % 2 == 0 else dstb0,
                           iB if seg % 2 == 0 else iA)

        def body(i, _):
            a = 3 * i
            gwait(rowsA, gA)
            dA = scat(a, db, rowsA, sA)
            gwait(rowsB, gB)
            dB = scat(a + 1, db, rowsB, sB)
            gwait(rowsC, gC)
            dC = scat(a + 2, db, rowsC, sC)
            dA.wait()
            gather(a + 3, sb, rowsA, gA)
            dB.wait()

            @pl.when(i < (SEGC - 1) // 3 - 1)
            def _():
                gather(a + 4, sb, rowsB, gB)
            dC.wait()

            @pl.when(i < (SEGC - 1) // 3 - 1)
            def _():
                gather(a + 5, sb, rowsC, gC)
            return 0
        lax.fori_loop(0, (SEGC - 1) // 3, body, 0)

        gwait(rowsA, gA)
        scat(SEGC - 1, db, rowsA, sA).wait()

    plsc.subcore_barrier()
    prev = [None, None]
    for k in range(NBLK // NS):
        b = k % 2
        buf, gsem, ssem = (rowsA, gA, sA) if b == 0 else (rowsB, gB, sB)
        if prev[b] is not None:
            prev[b].wait()
        r0 = (k * NS + s) * ZB
        pltpu.async_copy(acc.at[pl.ds(r0, ZB)], buf, gsem).wait()
        prev[b] = pltpu.async_copy(buf, s_hbm.at[c, pl.ds(r0, ZB)], ssem)
    prev[0].wait()
    prev[1].wait()

    @pl.when(s < ktail)
    def _():
        r0 = ((NBLK // NS) * NS + s) * ZB
        pltpu.async_copy(acc.at[pl.ds(r0, ZB)], rowsA, gA).wait()
        pltpu.async_copy(rowsA, s_hbm.at[c, pl.ds(r0, ZB)], sA).wait()


def _tc_mm_body(x_ref, w_ref, out_ref):
    out_ref[...] = jnp.dot(x_ref[...], w_ref[...],
                           preferred_element_type=jnp.float32)


def _tc_mm(x, W):
    rb = 1000
    return pl.pallas_call(
        _tc_mm_body,
        out_shape=jax.ShapeDtypeStruct((N_NODES, D), jnp.float32),
        grid=(N_NODES // rb,),
        in_specs=[
            pl.BlockSpec((rb, D), lambda i: (i, 0)),
            pl.BlockSpec((D, D), lambda i: (0, 0)),
        ],
        out_specs=pl.BlockSpec((rb, D), lambda i: (i, 0)),
    )(x, W)


def _tc_scale_body(h_ref, degp_ref, out_ref):
    deg = degp_ref[0, :, 0] + degp_ref[1, :, 0] + 1.0
    dinv = lax.rsqrt(deg)
    out_ref[...] = h_ref[...] * dinv[:, None]


def _tc_scale(h, degp):
    rb = 1000
    return pl.pallas_call(
        _tc_scale_body,
        out_shape=jax.ShapeDtypeStruct((N_NODES, D), jnp.float32),
        grid=(N_NODES // rb,),
        in_specs=[
            pl.BlockSpec((rb, D), lambda i: (i, 0)),
            pl.BlockSpec((NC, rb, 1), lambda i: (0, i, 0)),
        ],
        out_specs=pl.BlockSpec((rb, D), lambda i: (i, 0)),
    )(h, degp)


def _tc_final_body(s_ref, hp_ref, x_ref, b_ref, degp_ref, out_ref):
    deg = degp_ref[0, :, 0] + degp_ref[1, :, 0] + 1.0
    dinv = lax.rsqrt(deg)
    tot = s_ref[0] + s_ref[1] + hp_ref[...]
    out_ref[...] = jnp.maximum(tot * dinv[:, None] + b_ref[...] + x_ref[...], 0.0)


def _tc_final(S, hp, x, b2, degp):
    rb = 1000
    return pl.pallas_call(
        _tc_final_body,
        out_shape=jax.ShapeDtypeStruct((N_NODES, D), jnp.float32),
        grid=(N_NODES // rb,),
        in_specs=[
            pl.BlockSpec((NC, rb, D), lambda i: (0, i, 0)),
            pl.BlockSpec((rb, D), lambda i: (i, 0)),
            pl.BlockSpec((rb, D), lambda i: (i, 0)),
            pl.BlockSpec((1, D), lambda i: (0, 0)),
            pl.BlockSpec((NC, rb, 1), lambda i: (0, i, 0)),
        ],
        out_specs=pl.BlockSpec((rb, D), lambda i: (i, 0)),
    )(S, hp, x, b2, degp)


def kernel(x, edge_index, W, b):
    src = edge_index[0].astype(jnp.int32)
    dst = edge_index[1].astype(jnp.int32)

    h = _tc_mm(x, W)
    degp = _sc_deg(dst).reshape(NC, NPAD, 1)
    hp = _tc_scale(h, degp)
    S = _sc_scatter(hp, src, dst.reshape(NW, NSEG, SEGC, CH))
    return _tc_final(S, hp, x, b.reshape(1, D), degp)


# speedup vs baseline: 1.0559x; 1.0055x over previous
import functools

import jax
import jax.numpy as jnp
from jax import lax
from jax.experimental import pallas as pl
from jax.experimental.pallas import tpu as pltpu
from jax.experimental.pallas import tpu_sc as plsc

N_NODES = 10000
N_EDGES = 320000
D = 128

NC = 2
NS = 16
NW = NC * NS

CH = 80
EPT = N_EDGES // NW
NCHUNK = EPT // CH
NPAD = 10240
ZB = 80
NBLK = N_NODES // ZB
NSEG = 5
SEGC = NCHUNK // NSEG

_mesh = plsc.VectorSubcoreMesh(core_axis_name="c", subcore_axis_name="s")


HR = NPAD // D
RB8 = 8
NRB = HR // RB8


@functools.partial(
    pl.kernel,
    out_type=jax.ShapeDtypeStruct((NC, HR, D), jnp.float32),
    mesh=_mesh,
    scratch_types=[
        pltpu.VMEM((EPT,), jnp.int32),
        pltpu.VMEM((HR, D), jnp.float32),
        pltpu.VMEM((RB8, D), jnp.float32),
        pltpu.VMEM((RB8, D), jnp.float32),
        pltpu.VMEM((RB8, D), jnp.float32),
        pltpu.VMEM_SHARED((NS, HR, D), jnp.float32),
        pltpu.SemaphoreType.DMA,
        pltpu.SemaphoreType.DMA,
    ],
    compiler_params=pltpu.CompilerParams(needs_layout_passes=False),
)
def _sc_deg(dst_hbm, degp_hbm, dstb, hist, racc, rtmp0, rtmp1, slots, m0, m1):
    c = lax.axis_index("c")
    s = lax.axis_index("s")
    wid = c * NS + s

    dload = pltpu.async_copy(dst_hbm.at[pl.ds(wid * EPT, EPT)], dstb, m0)

    def zf(i, _):
        for k in range(D // 16):
            hist[i, pl.ds(k * 16, 16)] = jnp.zeros((16,), jnp.float32)
        return 0
    lax.fori_loop(0, HR, zf, 0)

    dload.wait()

    ones = jnp.ones((16,), jnp.float32)

    def body(i, _):
        for u in range(5):
            idx = dstb[pl.ds(i * 80 + u * 16, 16)]
            hi = lax.shift_right_logical(idx, 7)
            lo = lax.bitwise_and(idx, jnp.int32(D - 1))
            plsc.addupdate_scatter(hist, [hi, lo], ones)
        return 0
    lax.fori_loop(0, EPT // 80, body, 0)

    pltpu.sync_copy(hist, slots.at[s])
    plsc.subcore_barrier()

    @pl.when(s < NRB)
    def _():
        pltpu.sync_copy(slots.at[0, pl.ds(s * RB8, RB8)], racc)
        pend = pltpu.async_copy(slots.at[1, pl.ds(s * RB8, RB8)], rtmp0, m0)
        for r in range(1, NS):
            tmp = rtmp0 if r % 2 == 1 else rtmp1
            pend.wait()
            if r + 1 < NS:
                nxt = rtmp1 if r % 2 == 1 else rtmp0
                sem = m1 if r % 2 == 1 else m0
                pend = pltpu.async_copy(
                    slots.at[r + 1, pl.ds(s * RB8, RB8)], nxt, sem)

            def add(i, _):
                for k in range(D // 16):
                    racc[i, pl.ds(k * 16, 16)] = (
                        racc[i, pl.ds(k * 16, 16)] + tmp[i, pl.ds(k * 16, 16)])
                return 0
            lax.fori_loop(0, RB8, add, 0)
        pltpu.sync_copy(racc, degp_hbm.at[c, pl.ds(s * RB8, RB8)])


@functools.partial(
    pl.kernel,
    out_type=jax.ShapeDtypeStruct((NC, N_NODES, D), jnp.float32),
    mesh=_mesh,
    scratch_types=[
        pltpu.VMEM((SEGC * CH,), jnp.int32),
        pltpu.VMEM((SEGC * CH,), jnp.int32),
        pltpu.VMEM((SEGC, CH), jnp.int32),
        pltpu.VMEM((SEGC, CH), jnp.int32),
        pltpu.VMEM((CH, D), jnp.float32),
        pltpu.VMEM((CH, D), jnp.float32),
        pltpu.VMEM((CH, D), jnp.float32),
        pltpu.VMEM_SHARED((N_NODES, D), jnp.float32),
        pltpu.SemaphoreType.DMA,
        pltpu.SemaphoreType.DMA,
        pltpu.SemaphoreType.DMA,
        pltpu.SemaphoreType.DMA,
        pltpu.SemaphoreType.DMA,
        pltpu.SemaphoreType.DMA,
        pltpu.SemaphoreType.DMA,
        pltpu.SemaphoreType.DMA,
    ],
    compiler_params=pltpu.CompilerParams(needs_layout_passes=False),
)
def _sc_scatter(hp_hbm, src_hbm, dst_hbm, s_hbm,
                srcb0, srcb1, dstb0, dstb1, rowsA, rowsB, rowsC, acc,
                gA, gB, gC, sA, sB, sC, iA, iB):
    c = lax.axis_index("c")
    s = lax.axis_index("s")
    wid = c * NS + s
    ktail = NBLK - (NBLK // NS) * NS

    def loadseg(seg, sb, db, sem):
        d1 = pltpu.async_copy(
            src_hbm.at[pl.ds(wid * EPT + seg * SEGC * CH, SEGC * CH)], sb, sem)
        d2 = pltpu.async_copy(dst_hbm.at[wid, seg], db, sem)
        return (d1, d2)

    pend = loadseg(0, srcb0, dstb0, iA)

    def zfill(i, _):
        for k in range(D // 16):
            rowsC[i, pl.ds(k * 16, 16)] = jnp.zeros((16,), jnp.float32)
        return 0
    lax.fori_loop(0, ZB, zfill, 0)

    for k in range(NBLK // NS):
        pltpu.sync_copy(rowsC, acc.at[pl.ds((k * NS + s) * ZB, ZB)])

    @pl.when(s < ktail)
    def _():
        pltpu.sync_copy(rowsC, acc.at[pl.ds(((NBLK // NS) * NS + s) * ZB, ZB)])
    plsc.subcore_barrier()

    def gather(j, sb, buf, sem):
        return pltpu.async_copy(hp_hbm.at[sb.at[pl.ds(j * CH, CH)]], buf, sem)

    def gwait(buf, sem):
        pltpu.make_async_copy(hp_hbm.at[srcb0.at[pl.ds(0, CH)]], buf, sem).wait()

    def scat(j, db, buf, sem):
        return pltpu.async_copy(buf, acc.at[db.at[j]], sem, add=True)

    for seg in range(NSEG):
        sb, db = (srcb0, dstb0) if seg % 2 == 0 else (srcb1, dstb1)
        pend[0].wait()
        pend[1].wait()

        gather(0, sb, rowsA, gA)
        gather(1, sb, rowsB, gB)
        gather(2, sb, rowsC, gC)
        if seg + 1 < NSEG:
            pend = loadseg(seg + 1,
                           srcb1 if seg % 2 == 0 else srcb0,
                           dstb1 if seg % 2 == 0 else dstb0,
                           iB if seg % 2 == 0 else iA)

        def body(i, _):
            a = 3 * i
            gwait(rowsA, gA)
            dA = scat(a, db, rowsA, sA)
            gwait(rowsB, gB)
            dB = scat(a + 1, db, rowsB, sB)
            gwait(rowsC, gC)
            dC = scat(a + 2, db, rowsC, sC)
            dA.wait()
            gather(a + 3, sb, rowsA, gA)
            dB.wait()

            @pl.when(i < (SEGC - 1) // 3 - 1)
            def _():
                gather(a + 4, sb, rowsB, gB)
            dC.wait()

            @pl.when(i < (SEGC - 1) // 3 - 1)
            def _():
                gather(a + 5, sb, rowsC, gC)
            return 0
        lax.fori_loop(0, (SEGC - 1) // 3, body, 0)

        gwait(rowsA, gA)
        scat(SEGC - 1, db, rowsA, sA).wait()

    plsc.subcore_barrier()
    prev = [None, None]
    for k in range(NBLK // NS):
        b = k % 2
        buf, gsem, ssem = (rowsA, gA, sA) if b == 0 else (rowsB, gB, sB)
        if prev[b] is not None:
            prev[b].wait()
        r0 = (k * NS + s) * ZB
        pltpu.async_copy(acc.at[pl.ds(r0, ZB)], buf, gsem).wait()
        prev[b] = pltpu.async_copy(buf, s_hbm.at[c, pl.ds(r0, ZB)], ssem)
    prev[0].wait()
    prev[1].wait()

    @pl.when(s < ktail)
    def _():
        r0 = ((NBLK // NS) * NS + s) * ZB
        pltpu.async_copy(acc.at[pl.ds(r0, ZB)], rowsA, gA).wait()
        pltpu.async_copy(rowsA, s_hbm.at[c, pl.ds(r0, ZB)], sA).wait()


def _tc_mm_body(x_ref, w_ref, out_ref):
    out_ref[...] = jnp.dot(x_ref[...], w_ref[...],
                           preferred_element_type=jnp.float32)


def _tc_mm(x, W):
    rb = 1000
    return pl.pallas_call(
        _tc_mm_body,
        out_shape=jax.ShapeDtypeStruct((N_NODES, D), jnp.float32),
        grid=(N_NODES // rb,),
        in_specs=[
            pl.BlockSpec((rb, D), lambda i: (i, 0)),
            pl.BlockSpec((D, D), lambda i: (0, 0)),
        ],
        out_specs=pl.BlockSpec((rb, D), lambda i: (i, 0)),
    )(x, W)


def _tc_scale_body(h_ref, degp_ref, out_ref):
    deg = degp_ref[0, :, 0] + degp_ref[1, :, 0] + 1.0
    dinv = lax.rsqrt(deg)
    out_ref[...] = h_ref[...] * dinv[:, None]


def _tc_scale(h, degp):
    rb = 1000
    return pl.pallas_call(
        _tc_scale_body,
        out_shape=jax.ShapeDtypeStruct((N_NODES, D), jnp.float32),
        grid=(N_NODES // rb,),
        in_specs=[
            pl.BlockSpec((rb, D), lambda i: (i, 0)),
            pl.BlockSpec((NC, rb, 1), lambda i: (0, i, 0)),
        ],
        out_specs=pl.BlockSpec((rb, D), lambda i: (i, 0)),
    )(h, degp)


def _tc_final_body(s_ref, hp_ref, x_ref, b_ref, degp_ref, out_ref):
    deg = degp_ref[0, :, 0] + degp_ref[1, :, 0] + 1.0
    dinv = lax.rsqrt(deg)
    tot = s_ref[0] + s_ref[1] + hp_ref[...]
    out_ref[...] = jnp.maximum(tot * dinv[:, None] + b_ref[...] + x_ref[...], 0.0)


def _tc_final(S, hp, x, b2, degp):
    rb = 1000
    return pl.pallas_call(
        _tc_final_body,
        out_shape=jax.ShapeDtypeStruct((N_NODES, D), jnp.float32),
        grid=(N_NODES // rb,),
        in_specs=[
            pl.BlockSpec((NC, rb, D), lambda i: (0, i, 0)),
            pl.BlockSpec((rb, D), lambda i: (i, 0)),
            pl.BlockSpec((rb, D), lambda i: (i, 0)),
            pl.BlockSpec((1, D), lambda i: (0, 0)),
            pl.BlockSpec((NC, rb, 1), lambda i: (0, i, 0)),
        ],
        out_specs=pl.BlockSpec((rb, D), lambda i: (i, 0)),
    )(S, hp, x, b2, degp)


def kernel(x, edge_index, W, b):
    src = edge_index[0].astype(jnp.int32)
    dst = edge_index[1].astype(jnp.int32)

    h = _tc_mm(x, W)
    degp = _sc_deg(dst).reshape(NC, NPAD, 1)
    hp = _tc_scale(h, degp)
    S = _sc_scatter(hp, src, dst.reshape(NW, NSEG, SEGC, CH))
    return _tc_final(S, hp, x, b.reshape(1, D), degp)
